# baseline (speedup 1.0000x reference)
import functools

import jax
import jax.numpy as jnp
from jax import lax
from jax.experimental import pallas as pl
from jax.experimental.pallas import tpu as pltpu
from jax.experimental.pallas import tpu_sc as plsc

N = 10000
D = 128
E = 320000
R = 4

NW = 32
CHUNK = 64
CHUNK_SH = 6
NCH = 160
EPT = NCH * CHUNK
E_PAD = NW * EPT
SEG_CH = 16
NSEG = NCH // SEG_CH
TRASH = N
ACC_ROWS = 10112
SPT = ACC_ROWS // 16
ZPIECES = tuple((k * 64, 64) for k in range(9)) + ((576, 56),)
CCAP = SEG_CH * CHUNK + CHUNK

BLK = 400
GRID = N // BLK


def _mish(v):
    sp = jnp.log1p(jnp.exp(-jnp.abs(v))) + jnp.maximum(v, 0.0)
    return v * jnp.tanh(sp)


def _sc_body(x_hbm, src_hbm, dst_hbm, dropw_hbm, out_hbm,
             src_buf, dst_buf, ring, drop_tbl, rows_buf, csrc, cdst,
             sem0, sem1, acc):
    c = lax.axis_index("c")
    s = lax.axis_index("s")
    wid = c * 16 + s
    row0 = s * SPT

    pltpu.sync_copy(src_hbm.at[pl.ds(wid * EPT, EPT)], src_buf)
    pltpu.sync_copy(dst_hbm.at[pl.ds(wid * EPT, EPT)], dst_buf)
    pltpu.sync_copy(dropw_hbm, drop_tbl)

    zero16f = jnp.zeros((16,), jnp.float32)
    zero16i = jnp.zeros((16,), jnp.int32)
    trash16 = jnp.full((16,), TRASH, jnp.int32)

    def zb_body(j, carry):
        for k in range(8):
            rows_buf[0, j, pl.ds(k * 16, 16)] = zero16f
        return carry

    lax.fori_loop(0, CHUNK, zb_body, None)
    for off, nr in ZPIECES:
        pltpu.sync_copy(rows_buf.at[0, pl.ds(0, nr)],
                        acc.at[pl.ds(row0 + off, nr)])

    def fill_ring(slot, buf, eoff):
        for k in range(CHUNK // 16):
            ring[slot, pl.ds(k * 16, 16)] = buf[pl.ds(eoff + k * 16, 16)]

    plsc.subcore_barrier()

    def dma_body(i, carry):
        c0 = i * 2
        h0 = pltpu.async_copy(
            x_hbm.at[src_buf.at[pl.ds(c0 * CHUNK, CHUNK)]],
            rows_buf.at[0], sem0)
        h1 = pltpu.async_copy(
            x_hbm.at[src_buf.at[pl.ds(c0 * CHUNK + CHUNK, CHUNK)]],
            rows_buf.at[1], sem1)
        fill_ring(0, dst_buf, c0 * CHUNK)
        fill_ring(1, dst_buf, c0 * CHUNK + CHUNK)
        h0.wait()
        pltpu.sync_copy(rows_buf.at[0], acc.at[ring.at[0]], add=True)
        h1.wait()
        pltpu.sync_copy(rows_buf.at[1], acc.at[ring.at[1]], add=True)
        return carry

    lax.fori_loop(0, NCH // 2, dma_body, None)

    plsc.subcore_barrier()
    pltpu.sync_copy(acc.at[pl.ds(row0, SPT)],
                    out_hbm.at[0, c, pl.ds(row0, SPT)])

    for r in range(R):
        plsc.subcore_barrier()

        def seg_body(g, carry):
            base_e = g * SEG_CH * CHUNK

            def cbody(j, cnt):
                eoff = base_e + j * CHUNK
                for k in range(CHUNK // 16):
                    sv = src_buf[pl.ds(eoff + k * 16, 16)]
                    dv = dst_buf[pl.ds(eoff + k * 16, 16)]
                    wv = plsc.load_gather(drop_tbl, [
                        lax.shift_right_logical(sv, 3)])
                    sh = ((sv & 7) << 2) + r
                    bit = lax.shift_right_logical(wv, sh) & 1
                    m = bit != 0
                    plsc.store_compressed(csrc.at[pl.ds(cnt, 16)], sv,
                                          mask=m)
                    plsc.store_compressed(cdst.at[pl.ds(cnt, 16)], dv,
                                          mask=m)
                    cnt = cnt + jnp.sum(bit)
                return cnt

            cnt = lax.fori_loop(0, SEG_CH, cbody, jnp.int32(0))
            for k in range(CHUNK // 16):
                csrc[pl.ds(cnt + k * 16, 16)] = zero16i
                cdst[pl.ds(cnt + k * 16, 16)] = trash16
            nchc = lax.shift_right_logical(cnt + (CHUNK - 1), CHUNK_SH)

            def corr_body(j, carry2):
                h = pltpu.async_copy(
                    x_hbm.at[csrc.at[pl.ds(j * CHUNK, CHUNK)]],
                    rows_buf.at[0], sem0)
                fill_ring(0, cdst, j * CHUNK)
                h.wait()
                pltpu.sync_copy(rows_buf.at[0],
                                acc.at[ring.at[0]], add=True)
                return carry2

            lax.fori_loop(0, nchc, corr_body, None)
            return carry

        lax.fori_loop(0, NSEG, seg_body, None)

        plsc.subcore_barrier()
        pltpu.sync_copy(acc.at[pl.ds(row0, SPT)],
                        out_hbm.at[r + 1, c, pl.ds(row0, SPT)])


DROPW = 1256


def _sc_agg(x, src_p, dst_p, drop_words):
    mesh = plsc.VectorSubcoreMesh(core_axis_name="c", subcore_axis_name="s")
    fn = pl.kernel(
        _sc_body,
        out_type=jax.ShapeDtypeStruct((R + 1, 2, ACC_ROWS, D), jnp.float32),
        mesh=mesh,
        compiler_params=pltpu.CompilerParams(needs_layout_passes=False),
        scratch_types=[
            pltpu.VMEM((NCH * CHUNK,), jnp.int32),
            pltpu.VMEM((NCH * CHUNK,), jnp.int32),
            pltpu.VMEM((2, CHUNK), jnp.int32),
            pltpu.VMEM((DROPW,), jnp.int32),
            pltpu.VMEM((2, CHUNK, D), jnp.float32),
            pltpu.VMEM((CCAP,), jnp.int32),
            pltpu.VMEM((CCAP,), jnp.int32),
            pltpu.SemaphoreType.DMA,
            pltpu.SemaphoreType.DMA,
            pltpu.VMEM_SHARED((ACC_ROWS, D), jnp.float32),
        ],
    )
    return fn(x, src_p, dst_p, drop_words)


def _pack_drop(drop):
    bits4 = jnp.sum(drop.astype(jnp.int32) << jnp.arange(R)[:, None], axis=0)
    bits4 = jnp.concatenate(
        [bits4, jnp.zeros((DROPW * 8 - N,), jnp.int32)])
    return jnp.sum(bits4.reshape(DROPW, 8) << (4 * jnp.arange(8)), axis=1)


def _mlp_body(x_ref, w1_ref, b1_ref, w2_ref, b2_ref, o_ref):
    t = _mish(jnp.dot(x_ref[...], w1_ref[...],
                      preferred_element_type=jnp.float32) + b1_ref[...])
    o_ref[...] = jnp.dot(t, w2_ref[...],
                         preferred_element_type=jnp.float32) + b2_ref[...]


def _mlp_call(x, W1, b1, W2, b2):
    return pl.pallas_call(
        _mlp_body,
        grid=(GRID,),
        in_specs=[
            pl.BlockSpec((BLK, D), lambda i: (i, 0)),
            pl.BlockSpec((D, D), lambda i: (0, 0)),
            pl.BlockSpec((1, D), lambda i: (0, 0)),
            pl.BlockSpec((D, D), lambda i: (0, 0)),
            pl.BlockSpec((1, D), lambda i: (0, 0)),
        ],
        out_specs=pl.BlockSpec((BLK, D), lambda i: (i, 0)),
        out_shape=jax.ShapeDtypeStruct((N, D), jnp.float32),
    )(x, W1, b1.reshape(1, D), W2, b2.reshape(1, D))


def _post_body(x_ref, keep_ref, p_ref, wa_ref, ba_ref, wb_ref, bb_ref,
               s_ref, sm_ref, sq_ref):
    i = pl.program_id(0)
    xb = x_ref[...]
    kb = keep_ref[...]
    p = p_ref[...]
    acc = jnp.zeros((BLK, D), jnp.float32)
    base = p[0, 0] + p[0, 1]
    prev = base
    for r in range(R):
        cum = p[r + 1, 0] + p[r + 1, 1]
        h = xb * kb[:, r:r + 1] + (base - (cum - prev))
        prev = cum
        t = _mish(jnp.dot(h, wa_ref[...],
                          preferred_element_type=jnp.float32) + ba_ref[...])
        acc = acc + jnp.dot(t, wb_ref[...],
                            preferred_element_type=jnp.float32) + bb_ref[...]
    sb = acc * (1.0 / R) + xb
    s_ref[...] = sb

    @pl.when(i == 0)
    def _():
        sm_ref[...] = jnp.zeros((1, D), jnp.float32)
        sq_ref[...] = jnp.zeros((1, D), jnp.float32)

    sm_ref[...] += jnp.sum(sb, axis=0, keepdims=True)
    sq_ref[...] += jnp.sum(sb * sb, axis=0, keepdims=True)


def _post_call(x, keepT, P, Wa, ba, Wb, bb):
    return pl.pallas_call(
        _post_body,
        grid=(GRID,),
        in_specs=[
            pl.BlockSpec((BLK, D), lambda i: (i, 0)),
            pl.BlockSpec((BLK, R), lambda i: (i, 0)),
            pl.BlockSpec((R + 1, 2, BLK, D), lambda i: (0, 0, i, 0)),
            pl.BlockSpec((D, D), lambda i: (0, 0)),
            pl.BlockSpec((1, D), lambda i: (0, 0)),
            pl.BlockSpec((D, D), lambda i: (0, 0)),
            pl.BlockSpec((1, D), lambda i: (0, 0)),
        ],
        out_specs=[
            pl.BlockSpec((BLK, D), lambda i: (i, 0)),
            pl.BlockSpec((1, D), lambda i: (0, 0)),
            pl.BlockSpec((1, D), lambda i: (0, 0)),
        ],
        out_shape=[
            jax.ShapeDtypeStruct((N, D), jnp.float32),
            jax.ShapeDtypeStruct((1, D), jnp.float32),
            jax.ShapeDtypeStruct((1, D), jnp.float32),
        ],
    )(x, keepT, P, Wa, ba.reshape(1, D), Wb, bb.reshape(1, D))


def _bn_body(s_ref, sm_ref, sq_ref, g_ref, be_ref, o_ref):
    mu = sm_ref[...] * (1.0 / N)
    var = sq_ref[...] * (1.0 / N) - mu * mu
    inv = lax.rsqrt(var + 1e-5)
    o_ref[...] = _mish(g_ref[...] * (s_ref[...] - mu) * inv + be_ref[...])


def _bn_res_body(s_ref, sm_ref, sq_ref, g_ref, be_ref, res_ref, o_ref):
    mu = sm_ref[...] * (1.0 / N)
    var = sq_ref[...] * (1.0 / N) - mu * mu
    inv = lax.rsqrt(var + 1e-5)
    o_ref[...] = res_ref[...] + _mish(
        g_ref[...] * (s_ref[...] - mu) * inv + be_ref[...])


def _bn_call(sarr, sm, sq, g, be, res=None):
    specs = [
        pl.BlockSpec((BLK, D), lambda i: (i, 0)),
        pl.BlockSpec((1, D), lambda i: (0, 0)),
        pl.BlockSpec((1, D), lambda i: (0, 0)),
        pl.BlockSpec((1, D), lambda i: (0, 0)),
        pl.BlockSpec((1, D), lambda i: (0, 0)),
    ]
    args = [sarr, sm, sq, g.reshape(1, D), be.reshape(1, D)]
    body = _bn_body
    if res is not None:
        specs.append(pl.BlockSpec((BLK, D), lambda i: (i, 0)))
        args.append(res)
        body = _bn_res_body
    return pl.pallas_call(
        body,
        grid=(GRID,),
        in_specs=specs,
        out_specs=pl.BlockSpec((BLK, D), lambda i: (i, 0)),
        out_shape=jax.ShapeDtypeStruct((N, D), jnp.float32),
    )(*args)


def kernel(x, edge_index, drop0, drop1, Wn1, bn1, Wn2, bn2,
           W0a, b0a, W0b, b0b, g0, be0, W1a, b1a, W1b, b1b, g1, be1):
    drop0_w = _pack_drop(drop0)
    drop1_w = _pack_drop(drop1)
    keep0T = (1.0 - drop0.astype(jnp.float32)).T
    keep1T = (1.0 - drop1.astype(jnp.float32)).T

    pad = E_PAD - E
    src_p = jnp.concatenate([edge_index[0], jnp.zeros((pad,), jnp.int32)])
    dst_p = jnp.concatenate(
        [edge_index[1], jnp.full((pad,), TRASH, jnp.int32)])

    x_proj = _mlp_call(x, Wn1, bn1, Wn2, bn2)

    P0 = _sc_agg(x_proj, src_p, dst_p, drop0_w)
    s1, sm1, sq1 = _post_call(x_proj, keep0T, P0, W0a, b0a, W0b, b0b)
    h1 = _bn_call(s1, sm1, sq1, g0, be0)

    P1 = _sc_agg(h1, src_p, dst_p, drop1_w)
    s2, sm2, sq2 = _post_call(h1, keep1T, P1, W1a, b1a, W1b, b1b)
    out = _bn_call(s2, sm2, sq2, g1, be1, x_proj)
    return out


# speedup vs baseline: 1.3631x; 1.3631x over previous
import functools

import jax
import jax.numpy as jnp
from jax import lax
from jax.experimental import pallas as pl
from jax.experimental.pallas import tpu as pltpu
from jax.experimental.pallas import tpu_sc as plsc

N = 10000
D = 128
E = 320000
R = 4

NW = 32
CHUNK = 64
NCH = 159
EPT = NCH * CHUNK
E_PAD = NW * EPT
TRASH = N
ACC_ROWS = 10112
SPT = ACC_ROWS // 16
ZPIECES = tuple((k * 64, 64) for k in range(9)) + ((576, 56),)

BLK = 400
GRID = N // BLK


def _mish(v):
    sp = jnp.log1p(jnp.exp(-jnp.abs(v))) + jnp.maximum(v, 0.0)
    return v * jnp.tanh(sp)


def _sc_body(x_hbm, src_hbm, dst_hbm, dropw_hbm, out_hbm,
             src_buf, dst_buf, ring, drop_tbl, rows_buf,
             sem0, sem1, sem2, acc):
    c = lax.axis_index("c")
    s = lax.axis_index("s")
    wid = c * 16 + s
    row0 = s * SPT
    sems = (sem0, sem1, sem2)

    pltpu.sync_copy(src_hbm.at[pl.ds(wid * EPT, EPT)], src_buf)
    pltpu.sync_copy(dst_hbm.at[pl.ds(wid * EPT, EPT)], dst_buf)
    pltpu.sync_copy(dropw_hbm, drop_tbl)

    zero16f = jnp.zeros((16,), jnp.float32)
    trash16 = jnp.full((16,), TRASH, jnp.int32)

    def zb_body(j, carry):
        for k in range(8):
            rows_buf[0, j, pl.ds(k * 16, 16)] = zero16f
        return carry

    lax.fori_loop(0, CHUNK, zb_body, None)
    for off, nr in ZPIECES:
        pltpu.sync_copy(rows_buf.at[0, pl.ds(0, nr)],
                        acc.at[pl.ds(row0 + off, nr)])

    def fill_ring_masked(u, j, r):
        for k in range(CHUNK // 16):
            sv = src_buf[pl.ds(j * CHUNK + k * 16, 16)]
            dv = dst_buf[pl.ds(j * CHUNK + k * 16, 16)]
            wv = plsc.load_gather(drop_tbl, [
                lax.shift_right_logical(sv, 3)])
            bit = lax.shift_right_logical(wv, ((sv & 7) << 2) + r) & 1
            ring[u, pl.ds(k * 16, 16)] = jnp.where(bit != 0, trash16, dv)

    def gather(u, j):
        return pltpu.async_copy(
            x_hbm.at[src_buf.at[pl.ds(j * CHUNK, CHUNK)]],
            rows_buf.at[u], sems[u])

    def gwait(u):
        pltpu.make_async_copy(x_hbm.at[pl.ds(0, CHUNK)],
                              rows_buf.at[u], sems[u]).wait()

    for r in range(R):
        plsc.subcore_barrier()

        gather(0, 0)
        gather(1, 1)

        def pipe_body(i, carry):
            j3 = i * 3
            for u in range(3):
                j = j3 + u
                gwait(u)
                fill_ring_masked(u, j, r)
                gather((u + 2) % 3, j + 2)
                pltpu.sync_copy(rows_buf.at[u], acc.at[ring.at[u]],
                                add=True)
            return carry

        lax.fori_loop(0, NCH // 3 - 1, pipe_body, None)

        je = NCH - 3
        gwait(0)
        fill_ring_masked(0, je, r)
        gather(2, je + 2)
        pltpu.sync_copy(rows_buf.at[0], acc.at[ring.at[0]], add=True)
        gwait(1)
        fill_ring_masked(1, je + 1, r)
        pltpu.sync_copy(rows_buf.at[1], acc.at[ring.at[1]], add=True)
        gwait(2)
        fill_ring_masked(2, je + 2, r)
        pltpu.sync_copy(rows_buf.at[2], acc.at[ring.at[2]], add=True)

        plsc.subcore_barrier()
        for idx, (off, nr) in enumerate(ZPIECES):
            u = idx % 2
            if idx >= 2:
                poff, pnr = ZPIECES[idx - 2]
                pltpu.make_async_copy(
                    rows_buf.at[u, pl.ds(0, pnr)],
                    out_hbm.at[r, c, pl.ds(row0 + poff, pnr)],
                    sems[u]).wait()
            pltpu.sync_copy(acc.at[pl.ds(row0 + off, nr)],
                            rows_buf.at[u, pl.ds(0, nr)])
            pltpu.async_copy(rows_buf.at[u, pl.ds(0, nr)],
                             out_hbm.at[r, c, pl.ds(row0 + off, nr)],
                             sems[u])
        for idx in (8, 9):
            u = idx % 2
            off, nr = ZPIECES[idx]
            pltpu.make_async_copy(
                rows_buf.at[u, pl.ds(0, nr)],
                out_hbm.at[r, c, pl.ds(row0 + off, nr)],
                sems[u]).wait()


DROPW = 1256


def _sc_agg(x, src_p, dst_p, drop_words):
    mesh = plsc.VectorSubcoreMesh(core_axis_name="c", subcore_axis_name="s")
    fn = pl.kernel(
        _sc_body,
        out_type=jax.ShapeDtypeStruct((R, 2, ACC_ROWS, D), jnp.float32),
        mesh=mesh,
        compiler_params=pltpu.CompilerParams(needs_layout_passes=False),
        scratch_types=[
            pltpu.VMEM((NCH * CHUNK,), jnp.int32),
            pltpu.VMEM((NCH * CHUNK,), jnp.int32),
            pltpu.VMEM((3, CHUNK), jnp.int32),
            pltpu.VMEM((DROPW,), jnp.int32),
            pltpu.VMEM((3, CHUNK, D), jnp.float32),
            pltpu.SemaphoreType.DMA,
            pltpu.SemaphoreType.DMA,
            pltpu.SemaphoreType.DMA,
            pltpu.VMEM_SHARED((ACC_ROWS, D), jnp.float32),
        ],
    )
    return fn(x, src_p, dst_p, drop_words)


def _pack_drop(drop):
    bits4 = jnp.sum(drop.astype(jnp.int32) << jnp.arange(R)[:, None], axis=0)
    bits4 = jnp.concatenate(
        [bits4, jnp.zeros((DROPW * 8 - N,), jnp.int32)])
    return jnp.sum(bits4.reshape(DROPW, 8) << (4 * jnp.arange(8)), axis=1)


def _mlp_body(x_ref, w1_ref, b1_ref, w2_ref, b2_ref, o_ref):
    t = _mish(jnp.dot(x_ref[...], w1_ref[...],
                      preferred_element_type=jnp.float32) + b1_ref[...])
    o_ref[...] = jnp.dot(t, w2_ref[...],
                         preferred_element_type=jnp.float32) + b2_ref[...]


def _mlp_call(x, W1, b1, W2, b2):
    return pl.pallas_call(
        _mlp_body,
        grid=(GRID,),
        in_specs=[
            pl.BlockSpec((BLK, D), lambda i: (i, 0)),
            pl.BlockSpec((D, D), lambda i: (0, 0)),
            pl.BlockSpec((1, D), lambda i: (0, 0)),
            pl.BlockSpec((D, D), lambda i: (0, 0)),
            pl.BlockSpec((1, D), lambda i: (0, 0)),
        ],
        out_specs=pl.BlockSpec((BLK, D), lambda i: (i, 0)),
        out_shape=jax.ShapeDtypeStruct((N, D), jnp.float32),
    )(x, W1, b1.reshape(1, D), W2, b2.reshape(1, D))


def _post_body(x_ref, keep_ref, p_ref, wa_ref, ba_ref, wb_ref, bb_ref,
               s_ref, sm_ref, sq_ref):
    i = pl.program_id(0)
    xb = x_ref[...]
    kb = keep_ref[...]
    p = p_ref[...]
    acc = jnp.zeros((BLK, D), jnp.float32)
    prev = jnp.zeros((BLK, D), jnp.float32)
    for r in range(R):
        cum = p[r, 0] + p[r, 1]
        h = xb * kb[:, r:r + 1] + (cum - prev)
        prev = cum
        t = _mish(jnp.dot(h, wa_ref[...],
                          preferred_element_type=jnp.float32) + ba_ref[...])
        acc = acc + jnp.dot(t, wb_ref[...],
                            preferred_element_type=jnp.float32) + bb_ref[...]
    sb = acc * (1.0 / R) + xb
    s_ref[...] = sb

    @pl.when(i == 0)
    def _():
        sm_ref[...] = jnp.zeros((1, D), jnp.float32)
        sq_ref[...] = jnp.zeros((1, D), jnp.float32)

    sm_ref[...] += jnp.sum(sb, axis=0, keepdims=True)
    sq_ref[...] += jnp.sum(sb * sb, axis=0, keepdims=True)


def _post_call(x, keepT, P, Wa, ba, Wb, bb):
    return pl.pallas_call(
        _post_body,
        grid=(GRID,),
        in_specs=[
            pl.BlockSpec((BLK, D), lambda i: (i, 0)),
            pl.BlockSpec((BLK, R), lambda i: (i, 0)),
            pl.BlockSpec((R, 2, BLK, D), lambda i: (0, 0, i, 0)),
            pl.BlockSpec((D, D), lambda i: (0, 0)),
            pl.BlockSpec((1, D), lambda i: (0, 0)),
            pl.BlockSpec((D, D), lambda i: (0, 0)),
            pl.BlockSpec((1, D), lambda i: (0, 0)),
        ],
        out_specs=[
            pl.BlockSpec((BLK, D), lambda i: (i, 0)),
            pl.BlockSpec((1, D), lambda i: (0, 0)),
            pl.BlockSpec((1, D), lambda i: (0, 0)),
        ],
        out_shape=[
            jax.ShapeDtypeStruct((N, D), jnp.float32),
            jax.ShapeDtypeStruct((1, D), jnp.float32),
            jax.ShapeDtypeStruct((1, D), jnp.float32),
        ],
    )(x, keepT, P, Wa, ba.reshape(1, D), Wb, bb.reshape(1, D))


def _bn_body(s_ref, sm_ref, sq_ref, g_ref, be_ref, o_ref):
    mu = sm_ref[...] * (1.0 / N)
    var = sq_ref[...] * (1.0 / N) - mu * mu
    inv = lax.rsqrt(var + 1e-5)
    o_ref[...] = _mish(g_ref[...] * (s_ref[...] - mu) * inv + be_ref[...])


def _bn_res_body(s_ref, sm_ref, sq_ref, g_ref, be_ref, res_ref, o_ref):
    mu = sm_ref[...] * (1.0 / N)
    var = sq_ref[...] * (1.0 / N) - mu * mu
    inv = lax.rsqrt(var + 1e-5)
    o_ref[...] = res_ref[...] + _mish(
        g_ref[...] * (s_ref[...] - mu) * inv + be_ref[...])


def _bn_call(sarr, sm, sq, g, be, res=None):
    specs = [
        pl.BlockSpec((BLK, D), lambda i: (i, 0)),
        pl.BlockSpec((1, D), lambda i: (0, 0)),
        pl.BlockSpec((1, D), lambda i: (0, 0)),
        pl.BlockSpec((1, D), lambda i: (0, 0)),
        pl.BlockSpec((1, D), lambda i: (0, 0)),
    ]
    args = [sarr, sm, sq, g.reshape(1, D), be.reshape(1, D)]
    body = _bn_body
    if res is not None:
        specs.append(pl.BlockSpec((BLK, D), lambda i: (i, 0)))
        args.append(res)
        body = _bn_res_body
    return pl.pallas_call(
        body,
        grid=(GRID,),
        in_specs=specs,
        out_specs=pl.BlockSpec((BLK, D), lambda i: (i, 0)),
        out_shape=jax.ShapeDtypeStruct((N, D), jnp.float32),
    )(*args)


def kernel(x, edge_index, drop0, drop1, Wn1, bn1, Wn2, bn2,
           W0a, b0a, W0b, b0b, g0, be0, W1a, b1a, W1b, b1b, g1, be1):
    drop0_w = _pack_drop(drop0)
    drop1_w = _pack_drop(drop1)
    keep0T = (1.0 - drop0.astype(jnp.float32)).T
    keep1T = (1.0 - drop1.astype(jnp.float32)).T

    pad = E_PAD - E
    src_p = jnp.concatenate([edge_index[0], jnp.zeros((pad,), jnp.int32)])
    dst_p = jnp.concatenate(
        [edge_index[1], jnp.full((pad,), TRASH, jnp.int32)])

    x_proj = _mlp_call(x, Wn1, bn1, Wn2, bn2)

    P0 = _sc_agg(x_proj, src_p, dst_p, drop0_w)
    s1, sm1, sq1 = _post_call(x_proj, keep0T, P0, W0a, b0a, W0b, b0b)
    h1 = _bn_call(s1, sm1, sq1, g0, be0)

    P1 = _sc_agg(h1, src_p, dst_p, drop1_w)
    s2, sm2, sq2 = _post_call(h1, keep1T, P1, W1a, b1a, W1b, b1b)
    out = _bn_call(s2, sm2, sq2, g1, be1, x_proj)
    return out


# speedup vs baseline: 2.3561x; 1.7285x over previous
import functools

import jax
import jax.numpy as jnp
from jax import lax
from jax.experimental import pallas as pl
from jax.experimental.pallas import tpu as pltpu
from jax.experimental.pallas import tpu_sc as plsc

N = 10000
D = 128
E = 320000
R = 4

NW = 32
CHUNK = 64
NCH = 157
EPT = NCH * CHUNK
E_PAD = NW * EPT
TRASH = N
ACC_ROWS = 10112
SPT = ACC_ROWS // 16
ZPIECES = tuple((k * 64, 64) for k in range(9)) + ((576, 56),)

BLK = 400
GRID = N // BLK


def _mish(v):
    sp = jnp.log1p(jnp.exp(-jnp.abs(v))) + jnp.maximum(v, 0.0)
    return v * jnp.tanh(sp)


def _sc_body(x_hbm, src_hbm, dst3_hbm, dropw_hbm, out_hbm,
             src_buf, eff_buf, drop_tbl, rows_buf,
             sem0, sem1, acc):
    c = lax.axis_index("c")
    s = lax.axis_index("s")
    wid = c * 16 + s
    row0 = s * SPT

    pltpu.sync_copy(src_hbm.at[pl.ds(wid * EPT, EPT)], src_buf)
    pltpu.sync_copy(dropw_hbm, drop_tbl)

    zero16f = jnp.zeros((16,), jnp.float32)

    def zb_body(j, carry):
        for k in range(8):
            rows_buf[0, j, pl.ds(k * 16, 16)] = zero16f
        return carry

    lax.fori_loop(0, CHUNK, zb_body, None)
    for off, nr in ZPIECES:
        pltpu.sync_copy(rows_buf.at[0, pl.ds(0, nr)],
                        acc.at[pl.ds(row0 + off, nr)])

    trash16 = jnp.full((16,), TRASH, jnp.int32)

    for r in range(R):
        pltpu.sync_copy(dst3_hbm.at[wid], eff_buf)

        def eff_body(j, carry):
            base = j * CHUNK
            for k in range(CHUNK // 16):
                sv = src_buf[pl.ds(base + k * 16, 16)]
                dv = eff_buf[j, pl.ds(k * 16, 16)]
                wv = plsc.load_gather(drop_tbl, [
                    lax.shift_right_logical(sv, 3)])
                sh = ((sv & 7) << 2) + r
                bit = lax.shift_right_logical(wv, sh) & 1
                eff = jnp.where(bit != 0, trash16, dv)
                eff_buf[j, pl.ds(k * 16, 16)] = eff
            return carry

        lax.fori_loop(0, NCH, eff_body, None)

        plsc.subcore_barrier()

        pltpu.async_copy(
            x_hbm.at[src_buf.at[pl.ds(0, CHUNK)]], rows_buf.at[0], sem0)

        def gwait(u, sem):
            pltpu.make_async_copy(x_hbm.at[pl.ds(0, CHUNK)],
                                  rows_buf.at[u], sem).wait()

        def dma_body(i, carry):
            a = i * 2
            pltpu.async_copy(
                x_hbm.at[src_buf.at[pl.ds((a + 1) * CHUNK, CHUNK)]],
                rows_buf.at[1], sem1)
            gwait(0, sem0)
            pltpu.sync_copy(rows_buf.at[0],
                            acc.at[eff_buf.at[a]], add=True)
            pltpu.async_copy(
                x_hbm.at[src_buf.at[pl.ds((a + 2) * CHUNK, CHUNK)]],
                rows_buf.at[0], sem0)
            gwait(1, sem1)
            pltpu.sync_copy(rows_buf.at[1],
                            acc.at[eff_buf.at[a + 1]], add=True)
            return carry

        lax.fori_loop(0, (NCH - 1) // 2, dma_body, None)
        gwait(0, sem0)
        pltpu.sync_copy(rows_buf.at[0],
                        acc.at[eff_buf.at[NCH - 1]], add=True)

        plsc.subcore_barrier()

        pltpu.sync_copy(acc.at[pl.ds(row0, SPT)],
                        out_hbm.at[r, c, pl.ds(row0, SPT)])


DROPW = 1256


def _sc_agg(x, src_p, dst3, drop_words):
    mesh = plsc.VectorSubcoreMesh(core_axis_name="c", subcore_axis_name="s")
    fn = pl.kernel(
        _sc_body,
        out_type=jax.ShapeDtypeStruct((R, 2, ACC_ROWS, D), jnp.float32),
        mesh=mesh,
        compiler_params=pltpu.CompilerParams(needs_layout_passes=False),
        scratch_types=[
            pltpu.VMEM((NCH * CHUNK,), jnp.int32),
            pltpu.VMEM((NCH, CHUNK), jnp.int32),
            pltpu.VMEM((DROPW,), jnp.int32),
            pltpu.VMEM((2, CHUNK, D), jnp.float32),
            pltpu.SemaphoreType.DMA,
            pltpu.SemaphoreType.DMA,
            pltpu.VMEM_SHARED((ACC_ROWS, D), jnp.float32),
        ],
    )
    return fn(x, src_p, dst3, drop_words)


def _pack_drop(drop):
    bits4 = jnp.sum(drop.astype(jnp.int32) << jnp.arange(R)[:, None], axis=0)
    bits4 = jnp.concatenate(
        [bits4, jnp.zeros((DROPW * 8 - N,), jnp.int32)])
    return jnp.sum(bits4.reshape(DROPW, 8) << (4 * jnp.arange(8)), axis=1)


def _mlp_body(x_ref, w1_ref, b1_ref, w2_ref, b2_ref, o_ref):
    t = _mish(jnp.dot(x_ref[...], w1_ref[...],
                      preferred_element_type=jnp.float32) + b1_ref[...])
    o_ref[...] = jnp.dot(t, w2_ref[...],
                         preferred_element_type=jnp.float32) + b2_ref[...]


def _mlp_call(x, W1, b1, W2, b2):
    return pl.pallas_call(
        _mlp_body,
        grid=(GRID,),
        in_specs=[
            pl.BlockSpec((BLK, D), lambda i: (i, 0)),
            pl.BlockSpec((D, D), lambda i: (0, 0)),
            pl.BlockSpec((1, D), lambda i: (0, 0)),
            pl.BlockSpec((D, D), lambda i: (0, 0)),
            pl.BlockSpec((1, D), lambda i: (0, 0)),
        ],
        out_specs=pl.BlockSpec((BLK, D), lambda i: (i, 0)),
        out_shape=jax.ShapeDtypeStruct((N, D), jnp.float32),
    )(x, W1, b1.reshape(1, D), W2, b2.reshape(1, D))


def _post_body(x_ref, keep_ref, p_ref, wa_ref, ba_ref, wb_ref, bb_ref,
               s_ref, sm_ref, sq_ref):
    i = pl.program_id(0)
    xb = x_ref[...]
    kb = keep_ref[...]
    p = p_ref[...]
    acc = jnp.zeros((BLK, D), jnp.float32)
    prev = jnp.zeros((BLK, D), jnp.float32)
    for r in range(R):
        cum = p[r, 0] + p[r, 1]
        h = xb * kb[:, r:r + 1] + (cum - prev)
        prev = cum
        t = _mish(jnp.dot(h, wa_ref[...],
                          preferred_element_type=jnp.float32) + ba_ref[...])
        acc = acc + jnp.dot(t, wb_ref[...],
                            preferred_element_type=jnp.float32) + bb_ref[...]
    sb = acc * (1.0 / R) + xb
    s_ref[...] = sb

    @pl.when(i == 0)
    def _():
        sm_ref[...] = jnp.zeros((1, D), jnp.float32)
        sq_ref[...] = jnp.zeros((1, D), jnp.float32)

    sm_ref[...] += jnp.sum(sb, axis=0, keepdims=True)
    sq_ref[...] += jnp.sum(sb * sb, axis=0, keepdims=True)


def _post_call(x, keepT, P, Wa, ba, Wb, bb):
    return pl.pallas_call(
        _post_body,
        grid=(GRID,),
        in_specs=[
            pl.BlockSpec((BLK, D), lambda i: (i, 0)),
            pl.BlockSpec((BLK, R), lambda i: (i, 0)),
            pl.BlockSpec((R, 2, BLK, D), lambda i: (0, 0, i, 0)),
            pl.BlockSpec((D, D), lambda i: (0, 0)),
            pl.BlockSpec((1, D), lambda i: (0, 0)),
            pl.BlockSpec((D, D), lambda i: (0, 0)),
            pl.BlockSpec((1, D), lambda i: (0, 0)),
        ],
        out_specs=[
            pl.BlockSpec((BLK, D), lambda i: (i, 0)),
            pl.BlockSpec((1, D), lambda i: (0, 0)),
            pl.BlockSpec((1, D), lambda i: (0, 0)),
        ],
        out_shape=[
            jax.ShapeDtypeStruct((N, D), jnp.float32),
            jax.ShapeDtypeStruct((1, D), jnp.float32),
            jax.ShapeDtypeStruct((1, D), jnp.float32),
        ],
    )(x, keepT, P, Wa, ba.reshape(1, D), Wb, bb.reshape(1, D))


def _bn_body(s_ref, sm_ref, sq_ref, g_ref, be_ref, o_ref):
    mu = sm_ref[...] * (1.0 / N)
    var = sq_ref[...] * (1.0 / N) - mu * mu
    inv = lax.rsqrt(var + 1e-5)
    o_ref[...] = _mish(g_ref[...] * (s_ref[...] - mu) * inv + be_ref[...])


def _bn_res_body(s_ref, sm_ref, sq_ref, g_ref, be_ref, res_ref, o_ref):
    mu = sm_ref[...] * (1.0 / N)
    var = sq_ref[...] * (1.0 / N) - mu * mu
    inv = lax.rsqrt(var + 1e-5)
    o_ref[...] = res_ref[...] + _mish(
        g_ref[...] * (s_ref[...] - mu) * inv + be_ref[...])


def _bn_call(sarr, sm, sq, g, be, res=None):
    specs = [
        pl.BlockSpec((BLK, D), lambda i: (i, 0)),
        pl.BlockSpec((1, D), lambda i: (0, 0)),
        pl.BlockSpec((1, D), lambda i: (0, 0)),
        pl.BlockSpec((1, D), lambda i: (0, 0)),
        pl.BlockSpec((1, D), lambda i: (0, 0)),
    ]
    args = [sarr, sm, sq, g.reshape(1, D), be.reshape(1, D)]
    body = _bn_body
    if res is not None:
        specs.append(pl.BlockSpec((BLK, D), lambda i: (i, 0)))
        args.append(res)
        body = _bn_res_body
    return pl.pallas_call(
        body,
        grid=(GRID,),
        in_specs=specs,
        out_specs=pl.BlockSpec((BLK, D), lambda i: (i, 0)),
        out_shape=jax.ShapeDtypeStruct((N, D), jnp.float32),
    )(*args)


def kernel(x, edge_index, drop0, drop1, Wn1, bn1, Wn2, bn2,
           W0a, b0a, W0b, b0b, g0, be0, W1a, b1a, W1b, b1b, g1, be1):
    drop0_w = _pack_drop(drop0)
    drop1_w = _pack_drop(drop1)
    keep0T = (1.0 - drop0.astype(jnp.float32)).T
    keep1T = (1.0 - drop1.astype(jnp.float32)).T

    pad = E_PAD - E
    src_p = jnp.concatenate([edge_index[0], jnp.zeros((pad,), jnp.int32)])
    dst3 = jnp.concatenate(
        [edge_index[1], jnp.full((pad,), TRASH, jnp.int32)]
    ).reshape(NW, NCH, CHUNK)

    x_proj = _mlp_call(x, Wn1, bn1, Wn2, bn2)

    P0 = _sc_agg(x_proj, src_p, dst3, drop0_w)
    s1, sm1, sq1 = _post_call(x_proj, keep0T, P0, W0a, b0a, W0b, b0b)
    h1 = _bn_call(s1, sm1, sq1, g0, be0)

    P1 = _sc_agg(h1, src_p, dst3, drop1_w)
    s2, sm2, sq2 = _post_call(h1, keep1T, P1, W1a, b1a, W1b, b1b)
    out = _bn_call(s2, sm2, sq2, g1, be1, x_proj)
    return out
